# Initial kernel scaffold; baseline (speedup 1.0000x reference)
import jax
import jax.numpy as jnp
from jax.experimental import pallas as pl


def kernel(h, x, permutation_weights, expert_weights, bias):
    raise NotImplementedError("write your pallas kernel here")


# speedup vs baseline: 3.1726x; 3.1726x over previous
import functools

import jax
import jax.numpy as jnp
from jax.experimental import pallas as pl
from jax.experimental.pallas import tpu as pltpu

E, B, D, P, K = 16, 4096, 1024, 4, 2
BT = 256


def _body(x_ref, w_ref, b_ref, pw_ref, h_ref, y_ref, soft_ref, hard_ref):
    i = pl.program_id(0)
    nsteps = pl.num_programs(0)

    x = x_ref[...]
    w = w_ref[...]
    logits = jax.lax.dot_general(
        x, w, (((1,), (1,)), ((), ())),
        preferred_element_type=jnp.float32) + b_ref[...]

    col = jax.lax.broadcasted_iota(jnp.int32, (BT, E), 1)
    rank = jnp.zeros((BT, E), dtype=jnp.int32)
    for j in range(E):
        lj = logits[:, j:j + 1]
        gt = (lj > logits).astype(jnp.int32)
        eq = ((lj == logits) & (j < col)).astype(jnp.int32)
        rank = rank + gt + eq
    keep = rank < K

    scattered = jnp.where(keep, logits, 0.0)
    masked = jnp.where(scattered == 0.0, -jnp.inf, scattered)
    m = jnp.max(masked, axis=1, keepdims=True)
    ex = jnp.exp(masked - m)
    s = ex / jnp.sum(ex, axis=1, keepdims=True)

    perm = (pw_ref[0] + pw_ref[1] + pw_ref[2] + pw_ref[3]) * 0.25
    sp = jax.lax.dot_general(
        s, perm, (((1,), (0,)), ((), ())),
        preferred_element_type=jnp.float32)

    acc = h_ref[0] * sp[:, 0:1]
    for e in range(1, E):
        acc = acc + h_ref[e] * sp[:, e:e + 1]
    y_ref[...] = acc

    @pl.when(i == 0)
    def _init():
        soft_ref[...] = jnp.zeros_like(soft_ref)
        hard_ref[...] = jnp.zeros_like(hard_ref)

    soft_ref[...] += jnp.sum(sp, axis=0, keepdims=True)
    hard_ref[...] += jnp.sum((sp >= 1e-5).astype(jnp.float32), axis=0,
                             keepdims=True)

    @pl.when(i == nsteps - 1)
    def _finalize():
        soft_ref[...] = soft_ref[...] * (1.0 / B)
        hard_ref[...] = hard_ref[...] * (1.0 / B)


@functools.partial(jax.jit)
def kernel(h, x, permutation_weights, expert_weights, bias):
    bias2d = bias.reshape(1, E)
    grid = (B // BT,)
    y, soft, hard = pl.pallas_call(
        _body,
        grid=grid,
        in_specs=[
            pl.BlockSpec((BT, D), lambda i: (i, 0)),
            pl.BlockSpec((E, D), lambda i: (0, 0)),
            pl.BlockSpec((1, E), lambda i: (0, 0)),
            pl.BlockSpec((P, E, E), lambda i: (0, 0, 0)),
            pl.BlockSpec((E, BT, D), lambda i: (0, i, 0)),
        ],
        out_specs=[
            pl.BlockSpec((BT, D), lambda i: (i, 0)),
            pl.BlockSpec((1, E), lambda i: (0, 0)),
            pl.BlockSpec((1, E), lambda i: (0, 0)),
        ],
        out_shape=[
            jax.ShapeDtypeStruct((B, D), jnp.float32),
            jax.ShapeDtypeStruct((1, E), jnp.float32),
            jax.ShapeDtypeStruct((1, E), jnp.float32),
        ],
        compiler_params=pltpu.CompilerParams(
            dimension_semantics=("arbitrary",),
        ),
    )(x, expert_weights, bias2d, permutation_weights, h)
    return (y, soft.reshape(E, 1), hard.reshape(E, 1))
